# Initial kernel scaffold; baseline (speedup 1.0000x reference)
import jax
import jax.numpy as jnp
from jax.experimental import pallas as pl


def kernel(x, edge_index, edge_attr, batch, W_e, W1, b1, W2, b2, eps, W_out, b_out):
    raise NotImplementedError("write your pallas kernel here")


# speedup vs baseline: 3.3551x; 3.3551x over previous
import functools

import jax
import jax.numpy as jnp
from jax import lax
from jax.experimental import pallas as pl
from jax.experimental.pallas import tpu as pltpu
from jax.experimental.pallas import tpu_sc as plsc

N = 10000
E = 320000
D = 128
H = 256
DE = 16
L = 5
G = 256
FEAT = 256

NC = 2
NS = 16
NW = NC * NS
CH = 128
NCHUNK = E // CH
CHUNK_PER_W = NCHUNK // NW
EXTRA_W = NCHUNK % NW
ZROW = 80
NZCH = N // ZROW


def _sc_message_pass(h, e3, src3, dst3):
    mesh = plsc.VectorSubcoreMesh(core_axis_name="c", subcore_axis_name="s")

    @functools.partial(
        pl.kernel,
        out_type=jax.ShapeDtypeStruct((NC, N, D), jnp.float32),
        mesh=mesh,
        scratch_types=[
            pltpu.VMEM((1, CH), jnp.int32),
            pltpu.VMEM((1, CH), jnp.int32),
            pltpu.VMEM((CH, D), jnp.float32),
            pltpu.VMEM((CH, D), jnp.float32),
            pltpu.VMEM_SHARED((N, D), jnp.float32),
            pltpu.SemaphoreType.DMA,
        ],
    )
    def body(h_hbm, e_hbm, src_hbm, dst_hbm, out_hbm,
             srcc_v, dstc_v, hrow_v, mrow_v, agg_sh, sem):
        cid = lax.axis_index("c")
        sid = lax.axis_index("s")
        wid = cid * NS + sid

        def zero_row(r, carry):
            for c8 in range(D // 16):
                hrow_v[r, pl.ds(c8 * 16, 16)] = jnp.zeros((16,), jnp.float32)
            return carry
        lax.fori_loop(0, ZROW, zero_row, 0)
        for k in range((NZCH + NS - 1) // NS):
            zc = sid + NS * k
            @pl.when(zc < NZCH)
            def _():
                pltpu.sync_copy(hrow_v.at[pl.ds(0, ZROW)],
                                agg_sh.at[pl.ds(zc * ZROW, ZROW)])
        plsc.subcore_barrier()

        base = wid * CHUNK_PER_W + jnp.minimum(wid, EXTRA_W)
        cnt = CHUNK_PER_W + (wid < EXTRA_W).astype(jnp.int32)

        def chunk_body(k, carry):
            c = base + k
            pltpu.sync_copy(src_hbm.at[c], srcc_v)
            pltpu.sync_copy(dst_hbm.at[c], dstc_v)
            pltpu.async_copy(h_hbm.at[srcc_v.at[0]], hrow_v, sem).wait()
            pltpu.sync_copy(e_hbm.at[c], mrow_v)

            def compute_row(r, c2):
                for c8 in range(D // 16):
                    sl = pl.ds(c8 * 16, 16)
                    v = hrow_v[r, sl] + mrow_v[r, sl]
                    mrow_v[r, sl] = jnp.maximum(v, 0.0)
                return c2
            lax.fori_loop(0, CH, compute_row, 0)

            pltpu.sync_copy(mrow_v, agg_sh.at[dstc_v.at[0]], add=True)
            return carry
        lax.fori_loop(0, cnt, chunk_body, 0)
        plsc.subcore_barrier()

        for k in range((NZCH + NS - 1) // NS):
            zc = sid + NS * k
            @pl.when(zc < NZCH)
            def _():
                r0 = zc * ZROW
                pltpu.sync_copy(agg_sh.at[pl.ds(r0, ZROW)],
                                hrow_v.at[pl.ds(0, ZROW)])
                pltpu.sync_copy(hrow_v.at[pl.ds(0, ZROW)],
                                out_hbm.at[cid, pl.ds(r0, ZROW)])

    return body(h, e3, src3, dst3)


_EBLK = 8000


def _edge_encoder_body(ea_ref, we_ref, out_ref):
    out_ref[...] = jnp.dot(ea_ref[...], we_ref[...],
                           preferred_element_type=jnp.float32)


def _edge_encoder(edge_attr, W_e):
    grid = E // _EBLK
    return pl.pallas_call(
        _edge_encoder_body,
        grid=(grid,),
        in_specs=[
            pl.BlockSpec((_EBLK, DE), lambda i: (i, 0)),
            pl.BlockSpec((DE, D), lambda i: (0, 0)),
        ],
        out_specs=pl.BlockSpec((_EBLK, D), lambda i: (i, 0)),
        out_shape=jax.ShapeDtypeStruct((E, D), jnp.float32),
    )(edge_attr, W_e)


_RBLK = 2000


def _mlp_body(scale_ref, h_ref, agg_ref, w1_ref, b1_ref, w2_ref, b2_ref,
              out_ref, *, final_relu):
    u = scale_ref[0] * h_ref[...] + agg_ref[0] + agg_ref[1]
    t = jnp.dot(u, w1_ref[...], preferred_element_type=jnp.float32) + b1_ref[...]
    t = jnp.maximum(t, 0.0)
    z = jnp.dot(t, w2_ref[...], preferred_element_type=jnp.float32) + b2_ref[...]
    if final_relu:
        z = jnp.maximum(z, 0.0)
    out_ref[...] = z


def _mlp_layer(h, agg2, W1l, b1l, W2l, b2l, scale, final_relu):
    grid = N // _RBLK
    return pl.pallas_call(
        functools.partial(_mlp_body, final_relu=final_relu),
        grid=(grid,),
        in_specs=[
            pl.BlockSpec(memory_space=pltpu.SMEM),
            pl.BlockSpec((_RBLK, D), lambda i: (i, 0)),
            pl.BlockSpec((NC, _RBLK, D), lambda i: (0, i, 0)),
            pl.BlockSpec((D, H), lambda i: (0, 0)),
            pl.BlockSpec((1, H), lambda i: (0, 0)),
            pl.BlockSpec((H, D), lambda i: (0, 0)),
            pl.BlockSpec((1, D), lambda i: (0, 0)),
        ],
        out_specs=pl.BlockSpec((_RBLK, D), lambda i: (i, 0)),
        out_shape=jax.ShapeDtypeStruct((N, D), jnp.float32),
    )(scale, h, agg2, W1l, b1l, W2l, b2l)


def _mlp_last_body(scale_ref, h_ref, agg_ref, w1_ref, b1_ref, w2_ref, b2_ref,
                   batch_ref, out_ref, sums_ref, counts_ref):
    i = pl.program_id(0)
    u = scale_ref[0] * h_ref[...] + agg_ref[0] + agg_ref[1]
    t = jnp.dot(u, w1_ref[...], preferred_element_type=jnp.float32) + b1_ref[...]
    t = jnp.maximum(t, 0.0)
    z = jnp.dot(t, w2_ref[...], preferred_element_type=jnp.float32) + b2_ref[...]
    out_ref[...] = z

    b_blk = batch_ref[0, 0, :]
    iota_g = lax.broadcasted_iota(jnp.int32, (_RBLK, G), 1)
    onehot = (b_blk[:, None] == iota_g).astype(jnp.float32)
    part_sums = lax.dot_general(onehot, z, (((0,), (0,)), ((), ())),
                                preferred_element_type=jnp.float32)
    part_counts = jnp.sum(onehot, axis=0)[None, :]

    @pl.when(i == 0)
    def _():
        sums_ref[...] = jnp.zeros_like(sums_ref)
        counts_ref[...] = jnp.zeros_like(counts_ref)

    sums_ref[...] += part_sums
    counts_ref[...] += part_counts


def _mlp_last_layer(h, agg2, W1l, b1l, W2l, b2l, scale, batch2d):
    grid = N // _RBLK
    return pl.pallas_call(
        _mlp_last_body,
        grid=(grid,),
        in_specs=[
            pl.BlockSpec(memory_space=pltpu.SMEM),
            pl.BlockSpec((_RBLK, D), lambda i: (i, 0)),
            pl.BlockSpec((NC, _RBLK, D), lambda i: (0, i, 0)),
            pl.BlockSpec((D, H), lambda i: (0, 0)),
            pl.BlockSpec((1, H), lambda i: (0, 0)),
            pl.BlockSpec((H, D), lambda i: (0, 0)),
            pl.BlockSpec((1, D), lambda i: (0, 0)),
            pl.BlockSpec((1, 1, _RBLK), lambda i: (i, 0, 0)),
        ],
        out_specs=[
            pl.BlockSpec((_RBLK, D), lambda i: (i, 0)),
            pl.BlockSpec((G, D), lambda i: (0, 0)),
            pl.BlockSpec((1, G), lambda i: (0, 0)),
        ],
        out_shape=[
            jax.ShapeDtypeStruct((N, D), jnp.float32),
            jax.ShapeDtypeStruct((G, D), jnp.float32),
            jax.ShapeDtypeStruct((1, G), jnp.float32),
        ],
    )(scale, h, agg2, W1l, b1l, W2l, b2l, batch2d)


def _proj_body(sums_ref, counts_ref, wo_ref, bo_ref, out_ref):
    c = jnp.maximum(counts_ref[...], 1.0)
    mean = sums_ref[...] * (1.0 / c)[0, :, None]
    out_ref[...] = jnp.dot(mean, wo_ref[...],
                           preferred_element_type=jnp.float32) + bo_ref[...]


def _projection(sums, counts, W_out, b_out):
    return pl.pallas_call(
        _proj_body,
        in_specs=[
            pl.BlockSpec((G, D), lambda: (0, 0)),
            pl.BlockSpec((1, G), lambda: (0, 0)),
            pl.BlockSpec((D, FEAT), lambda: (0, 0)),
            pl.BlockSpec((1, FEAT), lambda: (0, 0)),
        ],
        out_specs=pl.BlockSpec((G, FEAT), lambda: (0, 0)),
        out_shape=jax.ShapeDtypeStruct((G, FEAT), jnp.float32),
    )(sums, counts, W_out, b_out)


def kernel(x, edge_index, edge_attr, batch, W_e, W1, b1, W2, b2, eps,
           W_out, b_out):
    src = edge_index[0].astype(jnp.int32)
    dst = edge_index[1].astype(jnp.int32)
    src3 = src.reshape(NCHUNK, 1, CH)
    dst3 = dst.reshape(NCHUNK, 1, CH)
    batch2d = batch.astype(jnp.int32).reshape(N // _RBLK, 1, _RBLK)

    e = _edge_encoder(edge_attr, W_e).reshape(NCHUNK, CH, D)

    h = x
    for l in range(L):
        agg2 = _sc_message_pass(h, e, src3, dst3)
        scale = (1.0 + eps[l]).reshape(1).astype(jnp.float32)
        if l < L - 1:
            h = _mlp_layer(h, agg2, W1[l], b1[l].reshape(1, H), W2[l],
                           b2[l].reshape(1, D), scale, final_relu=True)
        else:
            h, sums, counts = _mlp_last_layer(
                h, agg2, W1[l], b1[l].reshape(1, H), W2[l],
                b2[l].reshape(1, D), scale, batch2d)

    graph_embeds = _projection(sums, counts, W_out, b_out.reshape(1, FEAT))
    graph_mask = (counts[0] > 0.0)
    return graph_embeds, graph_mask, h


# speedup vs baseline: 5.7810x; 1.7231x over previous
import functools

import jax
import jax.numpy as jnp
from jax import lax
from jax.experimental import pallas as pl
from jax.experimental.pallas import tpu as pltpu
from jax.experimental.pallas import tpu_sc as plsc

N = 10000
E = 320000
D = 128
H = 256
DE = 16
L = 5
G = 256
FEAT = 256

NC = 2
NS = 16
NW = NC * NS
CH = 80
NCHUNK = E // CH
CHUNK_PER_W = NCHUNK // NW
ZROW = 80
NZCH = N // ZROW


def _sc_message_pass(h, e3, eidx3):
    mesh = plsc.VectorSubcoreMesh(core_axis_name="c", subcore_axis_name="s")

    @functools.partial(
        pl.kernel,
        out_type=jax.ShapeDtypeStruct((NC, N, D), jnp.float32),
        mesh=mesh,
        scratch_types=[
            pltpu.VMEM((2, 2, CH), jnp.int32),
            pltpu.VMEM((2, CH, D), jnp.float32),
            pltpu.VMEM((2, CH, D), jnp.float32),
            pltpu.VMEM_SHARED((N, D), jnp.float32),
            pltpu.SemaphoreType.DMA,
            pltpu.SemaphoreType.DMA,
            pltpu.SemaphoreType.DMA,
            pltpu.SemaphoreType.DMA,
        ],
    )
    def body(h_hbm, e_hbm, eidx_hbm, out_hbm,
             idx_v, hbuf, mbuf, agg_sh,
             gsem0, gsem1, esem0, esem1):
        cid = lax.axis_index("c")
        sid = lax.axis_index("s")
        wid = cid * NS + sid
        base = wid * CHUNK_PER_W

        def start_fetch(k, islot, hb, mb, gsem, esem):
            pltpu.async_copy(h_hbm.at[islot.at[0]], hb, gsem)
            pltpu.async_copy(e_hbm.at[base + k], mb, esem)

        def wait_fetch(k, islot, hb, mb, gsem, esem):
            pltpu.make_async_copy(h_hbm.at[islot.at[0]], hb, gsem).wait()
            pltpu.make_async_copy(e_hbm.at[base + k], mb, esem).wait()

        def compute(hb, mb):
            @plsc.parallel_loop(0, CH)
            def _(r):
                for c8 in range(D // 16):
                    sl = pl.ds(c8 * 16, 16)
                    mb[r, sl] = jnp.maximum(hb[r, sl] + mb[r, sl], 0.0)

        ivs = [idx_v.at[0], idx_v.at[1]]
        hbs = [hbuf.at[0], hbuf.at[1]]
        mbs = [mbuf.at[0], mbuf.at[1]]
        gsems = [gsem0, gsem1]
        esems = [esem0, esem1]

        @plsc.parallel_loop(0, ZROW)
        def _(r):
            for c8 in range(D // 16):
                hbuf[0, r, pl.ds(c8 * 16, 16)] = jnp.zeros((16,), jnp.float32)
        for k in range((NZCH + NS - 1) // NS):
            zc = sid + NS * k
            @pl.when(zc < NZCH)
            def _():
                pltpu.sync_copy(hbuf.at[0], agg_sh.at[pl.ds(zc * ZROW, ZROW)])
        plsc.subcore_barrier()

        pltpu.sync_copy(eidx_hbm.at[base], ivs[0])
        start_fetch(0, ivs[0], hbs[0], mbs[0], gsems[0], esems[0])

        def half(k, p):
            q = (p + 1) % 2
            @pl.when(k + 1 < CHUNK_PER_W)
            def _():
                pltpu.sync_copy(eidx_hbm.at[base + k + 1], ivs[q])
                start_fetch(k + 1, ivs[q], hbs[q], mbs[q],
                            gsems[q], esems[q])
            wait_fetch(k, ivs[p], hbs[p], mbs[p], gsems[p], esems[p])
            compute(hbs[p], mbs[p])
            pltpu.sync_copy(mbs[p], agg_sh.at[ivs[p].at[1]], add=True)

        def loop_body(k2, carry):
            half(2 * k2, 0)
            half(2 * k2 + 1, 1)
            return carry
        lax.fori_loop(0, CHUNK_PER_W // 2, loop_body, 0)
        half(jnp.int32(CHUNK_PER_W - 1), (CHUNK_PER_W - 1) % 2)
        plsc.subcore_barrier()

        for k in range((NZCH + NS - 1) // NS):
            zc = sid + NS * k
            @pl.when(zc < NZCH)
            def _():
                r0 = zc * ZROW
                pltpu.sync_copy(agg_sh.at[pl.ds(r0, ZROW)], hbuf.at[0])
                pltpu.sync_copy(hbuf.at[0], out_hbm.at[cid, pl.ds(r0, ZROW)])

    return body(h, e3, eidx3)


_EBLK = 8000


def _edge_encoder_body(ea_ref, we_ref, out_ref):
    out_ref[...] = jnp.dot(ea_ref[...], we_ref[...],
                           preferred_element_type=jnp.float32)


def _edge_encoder(edge_attr, W_e):
    grid = E // _EBLK
    return pl.pallas_call(
        _edge_encoder_body,
        grid=(grid,),
        in_specs=[
            pl.BlockSpec((_EBLK, DE), lambda i: (i, 0)),
            pl.BlockSpec((DE, D), lambda i: (0, 0)),
        ],
        out_specs=pl.BlockSpec((_EBLK, D), lambda i: (i, 0)),
        out_shape=jax.ShapeDtypeStruct((E, D), jnp.float32),
    )(edge_attr, W_e)


_RBLK = 2000


def _mlp_body(scale_ref, h_ref, agg_ref, w1_ref, b1_ref, w2_ref, b2_ref,
              out_ref, *, final_relu):
    u = scale_ref[0] * h_ref[...] + agg_ref[0] + agg_ref[1]
    t = jnp.dot(u, w1_ref[...], preferred_element_type=jnp.float32) + b1_ref[...]
    t = jnp.maximum(t, 0.0)
    z = jnp.dot(t, w2_ref[...], preferred_element_type=jnp.float32) + b2_ref[...]
    if final_relu:
        z = jnp.maximum(z, 0.0)
    out_ref[...] = z


def _mlp_layer(h, agg2, W1l, b1l, W2l, b2l, scale, final_relu):
    grid = N // _RBLK
    return pl.pallas_call(
        functools.partial(_mlp_body, final_relu=final_relu),
        grid=(grid,),
        in_specs=[
            pl.BlockSpec(memory_space=pltpu.SMEM),
            pl.BlockSpec((_RBLK, D), lambda i: (i, 0)),
            pl.BlockSpec((NC, _RBLK, D), lambda i: (0, i, 0)),
            pl.BlockSpec((D, H), lambda i: (0, 0)),
            pl.BlockSpec((1, H), lambda i: (0, 0)),
            pl.BlockSpec((H, D), lambda i: (0, 0)),
            pl.BlockSpec((1, D), lambda i: (0, 0)),
        ],
        out_specs=pl.BlockSpec((_RBLK, D), lambda i: (i, 0)),
        out_shape=jax.ShapeDtypeStruct((N, D), jnp.float32),
    )(scale, h, agg2, W1l, b1l, W2l, b2l)


def _mlp_last_body(scale_ref, h_ref, agg_ref, w1_ref, b1_ref, w2_ref, b2_ref,
                   batch_ref, out_ref, sums_ref, counts_ref):
    i = pl.program_id(0)
    u = scale_ref[0] * h_ref[...] + agg_ref[0] + agg_ref[1]
    t = jnp.dot(u, w1_ref[...], preferred_element_type=jnp.float32) + b1_ref[...]
    t = jnp.maximum(t, 0.0)
    z = jnp.dot(t, w2_ref[...], preferred_element_type=jnp.float32) + b2_ref[...]
    out_ref[...] = z

    b_blk = batch_ref[0, 0, :]
    iota_g = lax.broadcasted_iota(jnp.int32, (_RBLK, G), 1)
    onehot = (b_blk[:, None] == iota_g).astype(jnp.float32)
    part_sums = lax.dot_general(onehot, z, (((0,), (0,)), ((), ())),
                                preferred_element_type=jnp.float32)
    part_counts = jnp.sum(onehot, axis=0)[None, :]

    @pl.when(i == 0)
    def _():
        sums_ref[...] = jnp.zeros_like(sums_ref)
        counts_ref[...] = jnp.zeros_like(counts_ref)

    sums_ref[...] += part_sums
    counts_ref[...] += part_counts


def _mlp_last_layer(h, agg2, W1l, b1l, W2l, b2l, scale, batch2d):
    grid = N // _RBLK
    return pl.pallas_call(
        _mlp_last_body,
        grid=(grid,),
        in_specs=[
            pl.BlockSpec(memory_space=pltpu.SMEM),
            pl.BlockSpec((_RBLK, D), lambda i: (i, 0)),
            pl.BlockSpec((NC, _RBLK, D), lambda i: (0, i, 0)),
            pl.BlockSpec((D, H), lambda i: (0, 0)),
            pl.BlockSpec((1, H), lambda i: (0, 0)),
            pl.BlockSpec((H, D), lambda i: (0, 0)),
            pl.BlockSpec((1, D), lambda i: (0, 0)),
            pl.BlockSpec((1, 1, _RBLK), lambda i: (i, 0, 0)),
        ],
        out_specs=[
            pl.BlockSpec((_RBLK, D), lambda i: (i, 0)),
            pl.BlockSpec((G, D), lambda i: (0, 0)),
            pl.BlockSpec((1, G), lambda i: (0, 0)),
        ],
        out_shape=[
            jax.ShapeDtypeStruct((N, D), jnp.float32),
            jax.ShapeDtypeStruct((G, D), jnp.float32),
            jax.ShapeDtypeStruct((1, G), jnp.float32),
        ],
    )(scale, h, agg2, W1l, b1l, W2l, b2l, batch2d)


def _proj_body(sums_ref, counts_ref, wo_ref, bo_ref, out_ref):
    c = jnp.maximum(counts_ref[...], 1.0)
    mean = sums_ref[...] * (1.0 / c)[0, :, None]
    out_ref[...] = jnp.dot(mean, wo_ref[...],
                           preferred_element_type=jnp.float32) + bo_ref[...]


def _projection(sums, counts, W_out, b_out):
    return pl.pallas_call(
        _proj_body,
        in_specs=[
            pl.BlockSpec((G, D), lambda: (0, 0)),
            pl.BlockSpec((1, G), lambda: (0, 0)),
            pl.BlockSpec((D, FEAT), lambda: (0, 0)),
            pl.BlockSpec((1, FEAT), lambda: (0, 0)),
        ],
        out_specs=pl.BlockSpec((G, FEAT), lambda: (0, 0)),
        out_shape=jax.ShapeDtypeStruct((G, FEAT), jnp.float32),
    )(sums, counts, W_out, b_out)


def kernel(x, edge_index, edge_attr, batch, W_e, W1, b1, W2, b2, eps,
           W_out, b_out):
    eidx3 = (edge_index.astype(jnp.int32)
             .reshape(2, NCHUNK, CH).transpose(1, 0, 2))
    batch2d = batch.astype(jnp.int32).reshape(N // _RBLK, 1, _RBLK)

    e = _edge_encoder(edge_attr, W_e).reshape(NCHUNK, CH, D)

    h = x
    for l in range(L):
        agg2 = _sc_message_pass(h, e, eidx3)
        scale = (1.0 + eps[l]).reshape(1).astype(jnp.float32)
        if l < L - 1:
            h = _mlp_layer(h, agg2, W1[l], b1[l].reshape(1, H), W2[l],
                           b2[l].reshape(1, D), scale, final_relu=True)
        else:
            h, sums, counts = _mlp_last_layer(
                h, agg2, W1[l], b1[l].reshape(1, H), W2[l],
                b2[l].reshape(1, D), scale, batch2d)

    graph_embeds = _projection(sums, counts, W_out, b_out.reshape(1, FEAT))
    graph_mask = (counts[0] > 0.0)
    return graph_embeds, graph_mask, h


# speedup vs baseline: 5.7894x; 1.0014x over previous
import functools

import jax
import jax.numpy as jnp
from jax import lax
from jax.experimental import pallas as pl
from jax.experimental.pallas import tpu as pltpu
from jax.experimental.pallas import tpu_sc as plsc

N = 10000
E = 320000
D = 128
H = 256
DE = 16
L = 5
G = 256
FEAT = 256

NC = 2
NS = 16
NW = NC * NS
CH = 80
NCHUNK = E // CH
CHUNK_PER_W = NCHUNK // NW
ZROW = 80
NZCH = N // ZROW


def _sc_message_pass(h, e3, eidx3):
    mesh = plsc.VectorSubcoreMesh(core_axis_name="c", subcore_axis_name="s")

    @functools.partial(
        pl.kernel,
        out_type=jax.ShapeDtypeStruct((NC, N, D), jnp.float32),
        mesh=mesh,
        scratch_types=[
            pltpu.VMEM((2, 2, CH), jnp.int32),
            pltpu.VMEM((2, CH, D), jnp.float32),
            pltpu.VMEM((2, CH, D), jnp.float32),
            pltpu.VMEM_SHARED((N, D), jnp.float32),
            pltpu.SemaphoreType.DMA,
            pltpu.SemaphoreType.DMA,
            pltpu.SemaphoreType.DMA,
            pltpu.SemaphoreType.DMA,
            pltpu.SemaphoreType.DMA,
            pltpu.SemaphoreType.DMA,
        ],
    )
    def body(h_hbm, e_hbm, eidx_hbm, out_hbm,
             idx_v, hbuf, mbuf, agg_sh,
             gsem0, gsem1, esem0, esem1, ssem0, ssem1):
        cid = lax.axis_index("c")
        sid = lax.axis_index("s")
        wid = cid * NS + sid
        base = wid * CHUNK_PER_W

        def start_fetch(k, islot, hb, mb, gsem, esem):
            pltpu.async_copy(h_hbm.at[islot.at[0]], hb, gsem)
            pltpu.async_copy(e_hbm.at[base + k], mb, esem)

        def wait_fetch(k, islot, hb, mb, gsem, esem):
            pltpu.make_async_copy(h_hbm.at[islot.at[0]], hb, gsem).wait()
            pltpu.make_async_copy(e_hbm.at[base + k], mb, esem).wait()

        def compute(hb, mb):
            @plsc.parallel_loop(0, CH)
            def _(r):
                for c8 in range(D // 16):
                    sl = pl.ds(c8 * 16, 16)
                    mb[r, sl] = jnp.maximum(hb[r, sl] + mb[r, sl], 0.0)

        ivs = [idx_v.at[0], idx_v.at[1]]
        hbs = [hbuf.at[0], hbuf.at[1]]
        mbs = [mbuf.at[0], mbuf.at[1]]
        gsems = [gsem0, gsem1]
        esems = [esem0, esem1]
        ssems = [ssem0, ssem1]

        def start_scatter(mb, islot, ssem):
            pltpu.async_copy(mb, agg_sh.at[islot.at[1]], ssem, add=True)

        def drain_scatter(mb, islot, ssem):
            pltpu.make_async_copy(mb, agg_sh.at[islot.at[1]], ssem).wait()

        @plsc.parallel_loop(0, ZROW)
        def _(r):
            for c8 in range(D // 16):
                hbuf[0, r, pl.ds(c8 * 16, 16)] = jnp.zeros((16,), jnp.float32)
        for k in range((NZCH + NS - 1) // NS):
            zc = sid + NS * k
            @pl.when(zc < NZCH)
            def _():
                pltpu.sync_copy(hbuf.at[0], agg_sh.at[pl.ds(zc * ZROW, ZROW)])
        plsc.subcore_barrier()

        pltpu.sync_copy(eidx_hbm.at[base], ivs[0])
        start_fetch(0, ivs[0], hbs[0], mbs[0], gsems[0], esems[0])

        def half(k, p):
            q = (p + 1) % 2
            @pl.when(k + 1 < CHUNK_PER_W)
            def _():
                @pl.when(k >= 1)
                def _():
                    drain_scatter(mbs[q], ivs[q], ssems[q])
                pltpu.sync_copy(eidx_hbm.at[base + k + 1], ivs[q])
                start_fetch(k + 1, ivs[q], hbs[q], mbs[q],
                            gsems[q], esems[q])
            wait_fetch(k, ivs[p], hbs[p], mbs[p], gsems[p], esems[p])
            compute(hbs[p], mbs[p])
            start_scatter(mbs[p], ivs[p], ssems[p])

        def loop_body(k2, carry):
            half(2 * k2, 0)
            half(2 * k2 + 1, 1)
            return carry
        lax.fori_loop(0, CHUNK_PER_W // 2, loop_body, 0)
        half(jnp.int32(CHUNK_PER_W - 1), (CHUNK_PER_W - 1) % 2)
        drain_scatter(mbs[1], ivs[1], ssems[1])
        drain_scatter(mbs[0], ivs[0], ssems[0])
        plsc.subcore_barrier()

        for k in range((NZCH + NS - 1) // NS):
            zc = sid + NS * k
            @pl.when(zc < NZCH)
            def _():
                r0 = zc * ZROW
                pltpu.sync_copy(agg_sh.at[pl.ds(r0, ZROW)], hbuf.at[0])
                pltpu.sync_copy(hbuf.at[0], out_hbm.at[cid, pl.ds(r0, ZROW)])

    return body(h, e3, eidx3)


_EBLK = 8000


def _edge_encoder_body(ea_ref, we_ref, out_ref):
    out_ref[...] = jnp.dot(ea_ref[...], we_ref[...],
                           preferred_element_type=jnp.float32)


def _edge_encoder(edge_attr, W_e):
    grid = E // _EBLK
    return pl.pallas_call(
        _edge_encoder_body,
        grid=(grid,),
        in_specs=[
            pl.BlockSpec((_EBLK, DE), lambda i: (i, 0)),
            pl.BlockSpec((DE, D), lambda i: (0, 0)),
        ],
        out_specs=pl.BlockSpec((_EBLK, D), lambda i: (i, 0)),
        out_shape=jax.ShapeDtypeStruct((E, D), jnp.float32),
    )(edge_attr, W_e)


_RBLK = 2000


def _mlp_body(scale_ref, h_ref, agg_ref, w1_ref, b1_ref, w2_ref, b2_ref,
              out_ref, *, final_relu):
    u = scale_ref[0] * h_ref[...] + agg_ref[0] + agg_ref[1]
    t = jnp.dot(u, w1_ref[...], preferred_element_type=jnp.float32) + b1_ref[...]
    t = jnp.maximum(t, 0.0)
    z = jnp.dot(t, w2_ref[...], preferred_element_type=jnp.float32) + b2_ref[...]
    if final_relu:
        z = jnp.maximum(z, 0.0)
    out_ref[...] = z


def _mlp_layer(h, agg2, W1l, b1l, W2l, b2l, scale, final_relu):
    grid = N // _RBLK
    return pl.pallas_call(
        functools.partial(_mlp_body, final_relu=final_relu),
        grid=(grid,),
        in_specs=[
            pl.BlockSpec(memory_space=pltpu.SMEM),
            pl.BlockSpec((_RBLK, D), lambda i: (i, 0)),
            pl.BlockSpec((NC, _RBLK, D), lambda i: (0, i, 0)),
            pl.BlockSpec((D, H), lambda i: (0, 0)),
            pl.BlockSpec((1, H), lambda i: (0, 0)),
            pl.BlockSpec((H, D), lambda i: (0, 0)),
            pl.BlockSpec((1, D), lambda i: (0, 0)),
        ],
        out_specs=pl.BlockSpec((_RBLK, D), lambda i: (i, 0)),
        out_shape=jax.ShapeDtypeStruct((N, D), jnp.float32),
    )(scale, h, agg2, W1l, b1l, W2l, b2l)


def _mlp_last_body(scale_ref, h_ref, agg_ref, w1_ref, b1_ref, w2_ref, b2_ref,
                   batch_ref, out_ref, sums_ref, counts_ref):
    i = pl.program_id(0)
    u = scale_ref[0] * h_ref[...] + agg_ref[0] + agg_ref[1]
    t = jnp.dot(u, w1_ref[...], preferred_element_type=jnp.float32) + b1_ref[...]
    t = jnp.maximum(t, 0.0)
    z = jnp.dot(t, w2_ref[...], preferred_element_type=jnp.float32) + b2_ref[...]
    out_ref[...] = z

    b_blk = batch_ref[0, 0, :]
    iota_g = lax.broadcasted_iota(jnp.int32, (_RBLK, G), 1)
    onehot = (b_blk[:, None] == iota_g).astype(jnp.float32)
    part_sums = lax.dot_general(onehot, z, (((0,), (0,)), ((), ())),
                                preferred_element_type=jnp.float32)
    part_counts = jnp.sum(onehot, axis=0)[None, :]

    @pl.when(i == 0)
    def _():
        sums_ref[...] = jnp.zeros_like(sums_ref)
        counts_ref[...] = jnp.zeros_like(counts_ref)

    sums_ref[...] += part_sums
    counts_ref[...] += part_counts


def _mlp_last_layer(h, agg2, W1l, b1l, W2l, b2l, scale, batch2d):
    grid = N // _RBLK
    return pl.pallas_call(
        _mlp_last_body,
        grid=(grid,),
        in_specs=[
            pl.BlockSpec(memory_space=pltpu.SMEM),
            pl.BlockSpec((_RBLK, D), lambda i: (i, 0)),
            pl.BlockSpec((NC, _RBLK, D), lambda i: (0, i, 0)),
            pl.BlockSpec((D, H), lambda i: (0, 0)),
            pl.BlockSpec((1, H), lambda i: (0, 0)),
            pl.BlockSpec((H, D), lambda i: (0, 0)),
            pl.BlockSpec((1, D), lambda i: (0, 0)),
            pl.BlockSpec((1, 1, _RBLK), lambda i: (i, 0, 0)),
        ],
        out_specs=[
            pl.BlockSpec((_RBLK, D), lambda i: (i, 0)),
            pl.BlockSpec((G, D), lambda i: (0, 0)),
            pl.BlockSpec((1, G), lambda i: (0, 0)),
        ],
        out_shape=[
            jax.ShapeDtypeStruct((N, D), jnp.float32),
            jax.ShapeDtypeStruct((G, D), jnp.float32),
            jax.ShapeDtypeStruct((1, G), jnp.float32),
        ],
    )(scale, h, agg2, W1l, b1l, W2l, b2l, batch2d)


def _proj_body(sums_ref, counts_ref, wo_ref, bo_ref, out_ref):
    c = jnp.maximum(counts_ref[...], 1.0)
    mean = sums_ref[...] * (1.0 / c)[0, :, None]
    out_ref[...] = jnp.dot(mean, wo_ref[...],
                           preferred_element_type=jnp.float32) + bo_ref[...]


def _projection(sums, counts, W_out, b_out):
    return pl.pallas_call(
        _proj_body,
        in_specs=[
            pl.BlockSpec((G, D), lambda: (0, 0)),
            pl.BlockSpec((1, G), lambda: (0, 0)),
            pl.BlockSpec((D, FEAT), lambda: (0, 0)),
            pl.BlockSpec((1, FEAT), lambda: (0, 0)),
        ],
        out_specs=pl.BlockSpec((G, FEAT), lambda: (0, 0)),
        out_shape=jax.ShapeDtypeStruct((G, FEAT), jnp.float32),
    )(sums, counts, W_out, b_out)


def kernel(x, edge_index, edge_attr, batch, W_e, W1, b1, W2, b2, eps,
           W_out, b_out):
    eidx3 = (edge_index.astype(jnp.int32)
             .reshape(2, NCHUNK, CH).transpose(1, 0, 2))
    batch2d = batch.astype(jnp.int32).reshape(N // _RBLK, 1, _RBLK)

    e = _edge_encoder(edge_attr, W_e).reshape(NCHUNK, CH, D)

    h = x
    for l in range(L):
        agg2 = _sc_message_pass(h, e, eidx3)
        scale = (1.0 + eps[l]).reshape(1).astype(jnp.float32)
        if l < L - 1:
            h = _mlp_layer(h, agg2, W1[l], b1[l].reshape(1, H), W2[l],
                           b2[l].reshape(1, D), scale, final_relu=True)
        else:
            h, sums, counts = _mlp_last_layer(
                h, agg2, W1[l], b1[l].reshape(1, H), W2[l],
                b2[l].reshape(1, D), scale, batch2d)

    graph_embeds = _projection(sums, counts, W_out, b_out.reshape(1, FEAT))
    graph_mask = (counts[0] > 0.0)
    return graph_embeds, graph_mask, h


# speedup vs baseline: 7.3437x; 1.2685x over previous
import functools

import jax
import jax.numpy as jnp
from jax import lax
from jax.experimental import pallas as pl
from jax.experimental.pallas import tpu as pltpu
from jax.experimental.pallas import tpu_sc as plsc

N = 10000
E = 320000
D = 128
H = 256
DE = 16
L = 5
G = 256
FEAT = 256

NC = 2
NS = 16
NW = NC * NS
CH = 40
NCHUNK = E // CH
CHUNK_PER_W = NCHUNK // NW
NBUF = 4
NI = 8
IP = 4
ZROW = 40
NZCH = N // ZROW
ZPT = (NZCH + NS - 1) // NS


def _sc_message_pass(h, e3, eidx3):
    mesh = plsc.VectorSubcoreMesh(core_axis_name="c", subcore_axis_name="s")

    @functools.partial(
        pl.kernel,
        out_type=jax.ShapeDtypeStruct((NC, N, D), jnp.float32),
        mesh=mesh,
        scratch_types=[
            pltpu.VMEM((NI, 2, CH), jnp.int32),
            pltpu.VMEM((NBUF, CH, D), jnp.float32),
            pltpu.VMEM((NBUF, CH, D), jnp.float32),
            pltpu.VMEM_SHARED((N, D), jnp.float32),
            [pltpu.SemaphoreType.DMA] * NI,
            [pltpu.SemaphoreType.DMA] * NBUF,
            [pltpu.SemaphoreType.DMA] * NBUF,
            [pltpu.SemaphoreType.DMA] * NBUF,
        ],
    )
    def body(h_hbm, e_hbm, eidx_hbm, out_hbm,
             idx_v, hbuf, mbuf, agg_sh, isems, gsems, esems, ssems):
        cid = lax.axis_index("c")
        sid = lax.axis_index("s")
        wid = cid * NS + sid
        base = wid * CHUNK_PER_W

        ivs = [idx_v.at[i] for i in range(NI)]
        hbs = [hbuf.at[i] for i in range(NBUF)]
        mbs = [mbuf.at[i] for i in range(NBUF)]

        def start_idx(k, p8):
            pltpu.async_copy(eidx_hbm.at[base + k], ivs[p8], isems[p8])

        def wait_idx(k, p8):
            pltpu.make_async_copy(eidx_hbm.at[base + k], ivs[p8],
                                  isems[p8]).wait()

        def start_fetch(k, p8, p4):
            pltpu.async_copy(h_hbm.at[ivs[p8].at[0]], hbs[p4], gsems[p4])
            pltpu.async_copy(e_hbm.at[base + k], mbs[p4], esems[p4])

        def wait_fetch(k, p8, p4):
            pltpu.make_async_copy(h_hbm.at[ivs[p8].at[0]], hbs[p4],
                                  gsems[p4]).wait()
            pltpu.make_async_copy(e_hbm.at[base + k], mbs[p4],
                                  esems[p4]).wait()

        def compute(p4):
            hb, mb = hbs[p4], mbs[p4]
            @plsc.parallel_loop(0, CH)
            def _(r):
                for c8 in range(D // 16):
                    sl = pl.ds(c8 * 16, 16)
                    mb[r, sl] = jnp.maximum(hb[r, sl] + mb[r, sl], 0.0)

        def start_scatter(p8, p4):
            pltpu.async_copy(mbs[p4], agg_sh.at[ivs[p8].at[1]], ssems[p4],
                             add=True)

        def drain_scatter(p8, p4):
            pltpu.make_async_copy(mbs[p4], agg_sh.at[ivs[p8].at[1]],
                                  ssems[p4]).wait()

        for j in range(IP):
            start_idx(jnp.int32(j), j)

        @plsc.parallel_loop(0, ZROW)
        def _(r):
            for c8 in range(D // 16):
                hbuf[0, r, pl.ds(c8 * 16, 16)] = jnp.zeros((16,), jnp.float32)
        for k in range(ZPT):
            zc = sid + NS * k
            @pl.when(zc < NZCH)
            def _():
                pltpu.sync_copy(hbuf.at[0], agg_sh.at[pl.ds(zc * ZROW, ZROW)])
        plsc.subcore_barrier()

        for j in range(2):
            wait_idx(jnp.int32(j), j)
            start_fetch(jnp.int32(j), j, j)

        def half(k, p8, p4):
            d4 = (p4 + 2) % NBUF
            d8 = (p8 + 6) % NI
            f8 = (p8 + 2) % NI
            i8 = (p8 + IP) % NI
            @pl.when(k >= 2)
            def _():
                drain_scatter(d8, d4)
            @pl.when(k + IP < CHUNK_PER_W)
            def _():
                start_idx(k + IP, i8)
            @pl.when(k + 2 < CHUNK_PER_W)
            def _():
                wait_idx(k + 2, f8)
                start_fetch(k + 2, f8, d4)
            wait_fetch(k, p8, p4)
            compute(p4)
            start_scatter(p8, p4)

        def loop_body(k8, carry):
            for j in range(NI):
                half(NI * k8 + j, j, j % NBUF)
            return carry
        lax.fori_loop(0, CHUNK_PER_W // NI, loop_body, 0)
        for j in range(CHUNK_PER_W % NI):
            k = (CHUNK_PER_W // NI) * NI + j
            half(jnp.int32(k), k % NI, k % NBUF)

        for j in range(2):
            k = CHUNK_PER_W - 2 + j
            drain_scatter(k % NI, k % NBUF)
        plsc.subcore_barrier()

        for k in range(ZPT):
            zc = sid + NS * k
            if k >= 2:
                zcp = sid + NS * (k - 2)
                @pl.when(zcp < NZCH)
                def _():
                    pltpu.make_async_copy(hbuf.at[k % 2],
                                          out_hbm.at[cid, pl.ds(zcp * ZROW, ZROW)],
                                          gsems[k % 2]).wait()
            @pl.when(zc < NZCH)
            def _():
                pltpu.sync_copy(agg_sh.at[pl.ds(zc * ZROW, ZROW)],
                                hbuf.at[k % 2])
                pltpu.async_copy(hbuf.at[k % 2],
                                 out_hbm.at[cid, pl.ds(zc * ZROW, ZROW)],
                                 gsems[k % 2])
        for k in range(ZPT - 2, ZPT):
            zc = sid + NS * k
            @pl.when(zc < NZCH)
            def _():
                pltpu.make_async_copy(hbuf.at[k % 2],
                                      out_hbm.at[cid, pl.ds(zc * ZROW, ZROW)],
                                      gsems[k % 2]).wait()

    return body(h, e3, eidx3)


_EBLK = 8000


def _edge_encoder_body(ea_ref, we_ref, out_ref):
    out_ref[...] = jnp.dot(ea_ref[...], we_ref[...],
                           preferred_element_type=jnp.float32)


def _edge_encoder(edge_attr, W_e):
    grid = E // _EBLK
    return pl.pallas_call(
        _edge_encoder_body,
        grid=(grid,),
        in_specs=[
            pl.BlockSpec((_EBLK, DE), lambda i: (i, 0)),
            pl.BlockSpec((DE, D), lambda i: (0, 0)),
        ],
        out_specs=pl.BlockSpec((_EBLK, D), lambda i: (i, 0)),
        out_shape=jax.ShapeDtypeStruct((E, D), jnp.float32),
    )(edge_attr, W_e)


_RBLK = 2000


def _mlp_body(scale_ref, h_ref, agg_ref, w1_ref, b1_ref, w2_ref, b2_ref,
              out_ref, *, final_relu):
    u = scale_ref[0] * h_ref[...] + agg_ref[0] + agg_ref[1]
    t = jnp.dot(u, w1_ref[...], preferred_element_type=jnp.float32) + b1_ref[...]
    t = jnp.maximum(t, 0.0)
    z = jnp.dot(t, w2_ref[...], preferred_element_type=jnp.float32) + b2_ref[...]
    if final_relu:
        z = jnp.maximum(z, 0.0)
    out_ref[...] = z


def _mlp_layer(h, agg2, W1l, b1l, W2l, b2l, scale, final_relu):
    grid = N // _RBLK
    return pl.pallas_call(
        functools.partial(_mlp_body, final_relu=final_relu),
        grid=(grid,),
        in_specs=[
            pl.BlockSpec(memory_space=pltpu.SMEM),
            pl.BlockSpec((_RBLK, D), lambda i: (i, 0)),
            pl.BlockSpec((NC, _RBLK, D), lambda i: (0, i, 0)),
            pl.BlockSpec((D, H), lambda i: (0, 0)),
            pl.BlockSpec((1, H), lambda i: (0, 0)),
            pl.BlockSpec((H, D), lambda i: (0, 0)),
            pl.BlockSpec((1, D), lambda i: (0, 0)),
        ],
        out_specs=pl.BlockSpec((_RBLK, D), lambda i: (i, 0)),
        out_shape=jax.ShapeDtypeStruct((N, D), jnp.float32),
    )(scale, h, agg2, W1l, b1l, W2l, b2l)


def _mlp_last_body(scale_ref, h_ref, agg_ref, w1_ref, b1_ref, w2_ref, b2_ref,
                   batch_ref, out_ref, sums_ref, counts_ref):
    i = pl.program_id(0)
    u = scale_ref[0] * h_ref[...] + agg_ref[0] + agg_ref[1]
    t = jnp.dot(u, w1_ref[...], preferred_element_type=jnp.float32) + b1_ref[...]
    t = jnp.maximum(t, 0.0)
    z = jnp.dot(t, w2_ref[...], preferred_element_type=jnp.float32) + b2_ref[...]
    out_ref[...] = z

    b_blk = batch_ref[0, 0, :]
    iota_g = lax.broadcasted_iota(jnp.int32, (_RBLK, G), 1)
    onehot = (b_blk[:, None] == iota_g).astype(jnp.float32)
    part_sums = lax.dot_general(onehot, z, (((0,), (0,)), ((), ())),
                                preferred_element_type=jnp.float32)
    part_counts = jnp.sum(onehot, axis=0)[None, :]

    @pl.when(i == 0)
    def _():
        sums_ref[...] = jnp.zeros_like(sums_ref)
        counts_ref[...] = jnp.zeros_like(counts_ref)

    sums_ref[...] += part_sums
    counts_ref[...] += part_counts


def _mlp_last_layer(h, agg2, W1l, b1l, W2l, b2l, scale, batch2d):
    grid = N // _RBLK
    return pl.pallas_call(
        _mlp_last_body,
        grid=(grid,),
        in_specs=[
            pl.BlockSpec(memory_space=pltpu.SMEM),
            pl.BlockSpec((_RBLK, D), lambda i: (i, 0)),
            pl.BlockSpec((NC, _RBLK, D), lambda i: (0, i, 0)),
            pl.BlockSpec((D, H), lambda i: (0, 0)),
            pl.BlockSpec((1, H), lambda i: (0, 0)),
            pl.BlockSpec((H, D), lambda i: (0, 0)),
            pl.BlockSpec((1, D), lambda i: (0, 0)),
            pl.BlockSpec((1, 1, _RBLK), lambda i: (i, 0, 0)),
        ],
        out_specs=[
            pl.BlockSpec((_RBLK, D), lambda i: (i, 0)),
            pl.BlockSpec((G, D), lambda i: (0, 0)),
            pl.BlockSpec((1, G), lambda i: (0, 0)),
        ],
        out_shape=[
            jax.ShapeDtypeStruct((N, D), jnp.float32),
            jax.ShapeDtypeStruct((G, D), jnp.float32),
            jax.ShapeDtypeStruct((1, G), jnp.float32),
        ],
    )(scale, h, agg2, W1l, b1l, W2l, b2l, batch2d)


def _proj_body(sums_ref, counts_ref, wo_ref, bo_ref, out_ref):
    c = jnp.maximum(counts_ref[...], 1.0)
    mean = sums_ref[...] * (1.0 / c)[0, :, None]
    out_ref[...] = jnp.dot(mean, wo_ref[...],
                           preferred_element_type=jnp.float32) + bo_ref[...]


def _projection(sums, counts, W_out, b_out):
    return pl.pallas_call(
        _proj_body,
        in_specs=[
            pl.BlockSpec((G, D), lambda: (0, 0)),
            pl.BlockSpec((1, G), lambda: (0, 0)),
            pl.BlockSpec((D, FEAT), lambda: (0, 0)),
            pl.BlockSpec((1, FEAT), lambda: (0, 0)),
        ],
        out_specs=pl.BlockSpec((G, FEAT), lambda: (0, 0)),
        out_shape=jax.ShapeDtypeStruct((G, FEAT), jnp.float32),
    )(sums, counts, W_out, b_out)


def kernel(x, edge_index, edge_attr, batch, W_e, W1, b1, W2, b2, eps,
           W_out, b_out):
    eidx3 = (edge_index.astype(jnp.int32)
             .reshape(2, NCHUNK, CH).transpose(1, 0, 2))
    batch2d = batch.astype(jnp.int32).reshape(N // _RBLK, 1, _RBLK)

    e = _edge_encoder(edge_attr, W_e).reshape(NCHUNK, CH, D)

    h = x
    for l in range(L):
        agg2 = _sc_message_pass(h, e, eidx3)
        scale = (1.0 + eps[l]).reshape(1).astype(jnp.float32)
        if l < L - 1:
            h = _mlp_layer(h, agg2, W1[l], b1[l].reshape(1, H), W2[l],
                           b2[l].reshape(1, D), scale, final_relu=True)
        else:
            h, sums, counts = _mlp_last_layer(
                h, agg2, W1[l], b1[l].reshape(1, H), W2[l],
                b2[l].reshape(1, D), scale, batch2d)

    graph_embeds = _projection(sums, counts, W_out, b_out.reshape(1, FEAT))
    graph_mask = (counts[0] > 0.0)
    return graph_embeds, graph_mask, h
